# baseline (speedup 1.0000x reference)
import functools

import jax
import jax.numpy as jnp
from jax.experimental import pallas as pl
from jax.experimental.pallas import tpu as pltpu

ROWS = 320000
D_IN = 128
D_H = 256
D_M = 64
NUM_SEGMENTS = 512
BLK = 12800
NBLK = ROWS // BLK
S_TILE = 32
ACC_ROWS = NUM_SEGMENTS + S_TILE
_LN2 = 0.6931471805599453
_LOG2E = 1.4426950408889634


def _reduce_block(y, ids_row, acc_ref):
    act = jnp.log2(1.0 + jnp.exp2(y)).astype(jnp.bfloat16)
    s_lo = jnp.min(ids_row)
    s_hi = jnp.max(ids_row)
    base0 = (s_lo // S_TILE) * S_TILE
    iota = jax.lax.broadcasted_iota(jnp.int32, (S_TILE, BLK), 0) + base0
    oh = (iota == ids_row).astype(act.dtype)
    partial = jnp.dot(oh, act, preferred_element_type=jnp.float32)
    acc_ref[pl.ds(base0, S_TILE), :] += partial
    return act, base0, s_hi


def _reduce_overflow(act, ids_row, base0, s_hi, acc_ref):
    ntiles = (s_hi - base0) // S_TILE + 1

    def tile_body(t, _):
        base = base0 + t * S_TILE
        iota = jax.lax.broadcasted_iota(jnp.int32, (S_TILE, BLK), 0) + base
        oh = (iota == ids_row).astype(act.dtype)
        acc_ref[pl.ds(base, S_TILE), :] += jnp.dot(
            oh, act, preferred_element_type=jnp.float32)
        return 0

    jax.lax.fori_loop(1, ntiles, tile_body, 0)


def _body(h_ref, idsp_ref, idsc_ref, w1_ref, b1_ref, w2_ref, b2_ref, w3_ref,
          b3_ref, out_ref, acc_ref, y_ref):
    i = pl.program_id(0)

    @pl.when(i == 0)
    def _init():
        acc_ref[...] = jnp.zeros_like(acc_ref)

    act, base0, s_hi = _reduce_block(y_ref[...], idsp_ref[0], acc_ref)
    y_ref[...] = (jnp.dot(h_ref[...].astype(jnp.bfloat16), w1_ref[...],
                          preferred_element_type=jnp.float32) + b1_ref[...])

    @pl.when(s_hi - base0 >= S_TILE)
    def _overflow():
        _reduce_overflow(act, idsp_ref[0], base0, s_hi, acc_ref)

    @pl.when(i == NBLK - 1)
    def _last():
        actl, l_base0, l_hi = _reduce_block(y_ref[...], idsc_ref[0], acc_ref)

        @pl.when(l_hi - l_base0 >= S_TILE)
        def _overflow_l():
            _reduce_overflow(actl, idsc_ref[0], l_base0, l_hi, acc_ref)

        x = jax.nn.softplus(acc_ref[pl.ds(0, NUM_SEGMENTS), :] * jnp.float32(_LN2))
        x = jnp.dot(x, w2_ref[...], preferred_element_type=jnp.float32) + b2_ref[...]
        x = jax.nn.softplus(x)
        out_ref[...] = jnp.sum(x * w3_ref[...], axis=1, keepdims=True) + b3_ref[0, 0]


@jax.jit
def _run(h, ids_prev, ids_cur, W1, b1, W2, b2, w3row, b3):
    return pl.pallas_call(
        _body,
        grid=(NBLK,),
        in_specs=[
            pl.BlockSpec((BLK, D_IN), lambda i: (i, 0)),
            pl.BlockSpec((1, 1, BLK), lambda i: (i, 0, 0)),
            pl.BlockSpec((1, 1, BLK), lambda i: (i, 0, 0)),
            pl.BlockSpec((D_IN, D_H), lambda i: (0, 0)),
            pl.BlockSpec((1, D_H), lambda i: (0, 0)),
            pl.BlockSpec((D_H, D_M), lambda i: (0, 0)),
            pl.BlockSpec((1, D_M), lambda i: (0, 0)),
            pl.BlockSpec((1, D_M), lambda i: (0, 0)),
            pl.BlockSpec((1, 1), lambda i: (0, 0)),
        ],
        out_specs=pl.BlockSpec((NUM_SEGMENTS, 1), lambda i: (0, 0)),
        out_shape=jax.ShapeDtypeStruct((NUM_SEGMENTS, 1), jnp.float32),
        scratch_shapes=[
            pltpu.VMEM((ACC_ROWS, D_H), jnp.float32),
            pltpu.VMEM((BLK, D_H), jnp.float32),
        ],
    )(h, ids_prev, ids_cur, W1, b1, W2, b2, w3row, b3)


def kernel(h, batch, W1, b1, W2, b2, W3, b3):
    ids3 = batch.astype(jnp.int32).reshape(NBLK, 1, BLK)
    sentinel = jnp.full((1, 1, BLK), NUM_SEGMENTS, jnp.int32)
    ids_prev = jnp.concatenate([sentinel, ids3[:-1]], axis=0)
    log2e = jnp.float32(_LOG2E)
    return _run(h, ids_prev, ids3, (W1 * log2e).astype(jnp.bfloat16),
                (b1 * log2e).reshape(1, D_H),
                W2, b2.reshape(1, D_M), W3.reshape(1, D_M), b3.reshape(1, 1))


# speedup vs baseline: 1.0129x; 1.0129x over previous
import functools

import jax
import jax.numpy as jnp
from jax.experimental import pallas as pl
from jax.experimental.pallas import tpu as pltpu

ROWS = 320000
D_IN = 128
D_H = 256
D_M = 64
NUM_SEGMENTS = 512
BLK = 8000
NBLK = ROWS // BLK
S_TILE = 32
ACC_ROWS = NUM_SEGMENTS + S_TILE
_LN2 = 0.6931471805599453
_LOG2E = 1.4426950408889634


def _reduce_block(y, ids_row, acc_ref):
    act = jnp.log2(1.0 + jnp.exp2(y)).astype(jnp.bfloat16)
    s_lo = jnp.min(ids_row)
    s_hi = jnp.max(ids_row)
    base0 = (s_lo // S_TILE) * S_TILE
    iota = jax.lax.broadcasted_iota(jnp.int32, (S_TILE, BLK), 0) + base0
    oh = (iota == ids_row).astype(act.dtype)
    partial = jnp.dot(oh, act, preferred_element_type=jnp.float32)
    acc_ref[pl.ds(base0, S_TILE), :] += partial
    return act, base0, s_hi


def _reduce_overflow(act, ids_row, base0, s_hi, acc_ref):
    ntiles = (s_hi - base0) // S_TILE + 1

    def tile_body(t, _):
        base = base0 + t * S_TILE
        iota = jax.lax.broadcasted_iota(jnp.int32, (S_TILE, BLK), 0) + base
        oh = (iota == ids_row).astype(act.dtype)
        acc_ref[pl.ds(base, S_TILE), :] += jnp.dot(
            oh, act, preferred_element_type=jnp.float32)
        return 0

    jax.lax.fori_loop(1, ntiles, tile_body, 0)


def _body(h_ref, idsp_ref, idsc_ref, w1_ref, b1_ref, w2_ref, b2_ref, w3_ref,
          b3_ref, out_ref, acc_ref, y_ref):
    i = pl.program_id(0)

    @pl.when(i == 0)
    def _init():
        acc_ref[...] = jnp.zeros_like(acc_ref)

    act, base0, s_hi = _reduce_block(y_ref[...], idsp_ref[0], acc_ref)
    y_ref[...] = (jnp.dot(h_ref[...].astype(jnp.bfloat16), w1_ref[...],
                          preferred_element_type=jnp.float32) + b1_ref[...])

    @pl.when(s_hi - base0 >= S_TILE)
    def _overflow():
        _reduce_overflow(act, idsp_ref[0], base0, s_hi, acc_ref)

    @pl.when(i == NBLK - 1)
    def _last():
        actl, l_base0, l_hi = _reduce_block(y_ref[...], idsc_ref[0], acc_ref)

        @pl.when(l_hi - l_base0 >= S_TILE)
        def _overflow_l():
            _reduce_overflow(actl, idsc_ref[0], l_base0, l_hi, acc_ref)

        x = jax.nn.softplus(acc_ref[pl.ds(0, NUM_SEGMENTS), :] * jnp.float32(_LN2))
        x = jnp.dot(x, w2_ref[...], preferred_element_type=jnp.float32) + b2_ref[...]
        x = jax.nn.softplus(x)
        out_ref[...] = jnp.sum(x * w3_ref[...], axis=1, keepdims=True) + b3_ref[0, 0]


@jax.jit
def _run(h, ids_prev, ids_cur, W1, b1, W2, b2, w3row, b3):
    return pl.pallas_call(
        _body,
        grid=(NBLK,),
        in_specs=[
            pl.BlockSpec((BLK, D_IN), lambda i: (i, 0)),
            pl.BlockSpec((1, 1, BLK), lambda i: (i, 0, 0)),
            pl.BlockSpec((1, 1, BLK), lambda i: (i, 0, 0)),
            pl.BlockSpec((D_IN, D_H), lambda i: (0, 0)),
            pl.BlockSpec((1, D_H), lambda i: (0, 0)),
            pl.BlockSpec((D_H, D_M), lambda i: (0, 0)),
            pl.BlockSpec((1, D_M), lambda i: (0, 0)),
            pl.BlockSpec((1, D_M), lambda i: (0, 0)),
            pl.BlockSpec((1, 1), lambda i: (0, 0)),
        ],
        out_specs=pl.BlockSpec((NUM_SEGMENTS, 1), lambda i: (0, 0)),
        out_shape=jax.ShapeDtypeStruct((NUM_SEGMENTS, 1), jnp.float32),
        scratch_shapes=[
            pltpu.VMEM((ACC_ROWS, D_H), jnp.float32),
            pltpu.VMEM((BLK, D_H), jnp.float32),
        ],
    )(h, ids_prev, ids_cur, W1, b1, W2, b2, w3row, b3)


def kernel(h, batch, W1, b1, W2, b2, W3, b3):
    ids3 = batch.astype(jnp.int32).reshape(NBLK, 1, BLK)
    sentinel = jnp.full((1, 1, BLK), NUM_SEGMENTS, jnp.int32)
    ids_prev = jnp.concatenate([sentinel, ids3[:-1]], axis=0)
    log2e = jnp.float32(_LOG2E)
    return _run(h, ids_prev, ids3, (W1 * log2e).astype(jnp.bfloat16),
                (b1 * log2e).reshape(1, D_H),
                W2, b2.reshape(1, D_M), W3.reshape(1, D_M), b3.reshape(1, 1))


# speedup vs baseline: 1.0184x; 1.0055x over previous
import functools

import jax
import jax.numpy as jnp
from jax.experimental import pallas as pl
from jax.experimental.pallas import tpu as pltpu

ROWS = 320000
D_IN = 128
D_H = 256
D_M = 64
NUM_SEGMENTS = 512
BLK = 6400
NBLK = ROWS // BLK
S_TILE = 32
ACC_ROWS = NUM_SEGMENTS + S_TILE
_LN2 = 0.6931471805599453
_LOG2E = 1.4426950408889634


def _reduce_block(y, ids_row, acc_ref):
    act = jnp.log2(1.0 + jnp.exp2(y))
    s_lo = jnp.min(ids_row)
    s_hi = jnp.max(ids_row)
    base0 = (s_lo // S_TILE) * S_TILE
    iota = jax.lax.broadcasted_iota(jnp.int32, (S_TILE, BLK), 0) + base0
    oh = (iota == ids_row).astype(act.dtype)
    partial = jnp.dot(oh, act, preferred_element_type=jnp.float32)
    acc_ref[pl.ds(base0, S_TILE), :] += partial
    return act, base0, s_hi


def _reduce_overflow(act, ids_row, base0, s_hi, acc_ref):
    ntiles = (s_hi - base0) // S_TILE + 1

    def tile_body(t, _):
        base = base0 + t * S_TILE
        iota = jax.lax.broadcasted_iota(jnp.int32, (S_TILE, BLK), 0) + base
        oh = (iota == ids_row).astype(act.dtype)
        acc_ref[pl.ds(base, S_TILE), :] += jnp.dot(
            oh, act, preferred_element_type=jnp.float32)
        return 0

    jax.lax.fori_loop(1, ntiles, tile_body, 0)


def _body(h_ref, idsp_ref, idsc_ref, w1_ref, b1_ref, w2_ref, b2_ref, w3_ref,
          b3_ref, out_ref, acc_ref, y_ref):
    i = pl.program_id(0)

    @pl.when(i == 0)
    def _init():
        acc_ref[...] = jnp.zeros_like(acc_ref)

    act, base0, s_hi = _reduce_block(y_ref[...], idsp_ref[0], acc_ref)
    y_ref[...] = (jnp.dot(h_ref[...].astype(jnp.bfloat16), w1_ref[...],
                          preferred_element_type=jnp.float32) + b1_ref[...])

    @pl.when(s_hi - base0 >= S_TILE)
    def _overflow():
        _reduce_overflow(act, idsp_ref[0], base0, s_hi, acc_ref)

    @pl.when(i == NBLK - 1)
    def _last():
        actl, l_base0, l_hi = _reduce_block(y_ref[...], idsc_ref[0], acc_ref)

        @pl.when(l_hi - l_base0 >= S_TILE)
        def _overflow_l():
            _reduce_overflow(actl, idsc_ref[0], l_base0, l_hi, acc_ref)

        x = jax.nn.softplus(acc_ref[pl.ds(0, NUM_SEGMENTS), :] * jnp.float32(_LN2))
        x = jnp.dot(x, w2_ref[...], preferred_element_type=jnp.float32) + b2_ref[...]
        x = jax.nn.softplus(x)
        out_ref[...] = jnp.sum(x * w3_ref[...], axis=1, keepdims=True) + b3_ref[0, 0]


@jax.jit
def _run(h, ids_prev, ids_cur, W1, b1, W2, b2, w3row, b3):
    return pl.pallas_call(
        _body,
        grid=(NBLK,),
        in_specs=[
            pl.BlockSpec((BLK, D_IN), lambda i: (i, 0)),
            pl.BlockSpec((1, 1, BLK), lambda i: (i, 0, 0)),
            pl.BlockSpec((1, 1, BLK), lambda i: (i, 0, 0)),
            pl.BlockSpec((D_IN, D_H), lambda i: (0, 0)),
            pl.BlockSpec((1, D_H), lambda i: (0, 0)),
            pl.BlockSpec((D_H, D_M), lambda i: (0, 0)),
            pl.BlockSpec((1, D_M), lambda i: (0, 0)),
            pl.BlockSpec((1, D_M), lambda i: (0, 0)),
            pl.BlockSpec((1, 1), lambda i: (0, 0)),
        ],
        out_specs=pl.BlockSpec((NUM_SEGMENTS, 1), lambda i: (0, 0)),
        out_shape=jax.ShapeDtypeStruct((NUM_SEGMENTS, 1), jnp.float32),
        scratch_shapes=[
            pltpu.VMEM((ACC_ROWS, D_H), jnp.float32),
            pltpu.VMEM((BLK, D_H), jnp.float32),
        ],
    )(h, ids_prev, ids_cur, W1, b1, W2, b2, w3row, b3)


def kernel(h, batch, W1, b1, W2, b2, W3, b3):
    ids3 = batch.astype(jnp.int32).reshape(NBLK, 1, BLK)
    sentinel = jnp.full((1, 1, BLK), NUM_SEGMENTS, jnp.int32)
    ids_prev = jnp.concatenate([sentinel, ids3[:-1]], axis=0)
    log2e = jnp.float32(_LOG2E)
    return _run(h, ids_prev, ids3, (W1 * log2e).astype(jnp.bfloat16),
                (b1 * log2e).reshape(1, D_H),
                W2, b2.reshape(1, D_M), W3.reshape(1, D_M), b3.reshape(1, 1))
